# Initial kernel scaffold; baseline (speedup 1.0000x reference)
import jax
import jax.numpy as jnp
from jax.experimental import pallas as pl


def kernel(input, embed, pos_weight):
    raise NotImplementedError("write your pallas kernel here")


# speedup vs baseline: 1.0889x; 1.0889x over previous
import functools

import jax
import jax.numpy as jnp
from jax.experimental import pallas as pl

DIM = 16
N_EMBED = 1024
BLOCK = 1024
N_POS_BLOCKS = 8
N_BATCH = 8


def _vq_body(x_ref, pos_ref, emb_ref, embt_ref, idx_ref, quant_ref, loss_ref):
    p = pl.program_id(0)
    b = pl.program_id(1)

    x = x_ref[...]
    q = x + pos_ref[...]
    q = x + (q - x)

    emb = emb_ref[...]
    mm = jax.lax.dot_general(
        q, emb, (((1,), (0,)), ((), ())),
        preferred_element_type=jnp.float32)
    rowsum = jnp.sum(q * q, axis=1, keepdims=True)
    colsum = jnp.sum(emb * emb, axis=0, keepdims=True)
    dist = rowsum - 2.0 * mm + colsum

    m = jnp.min(dist, axis=1, keepdims=True)
    lanes = jax.lax.broadcasted_iota(jnp.int32, dist.shape, 1)
    idx = jnp.min(jnp.where(dist == m, lanes, jnp.int32(2**30)),
                  axis=1, keepdims=True)
    idx_ref[...] = idx

    onehot = (lanes == idx).astype(jnp.float32)
    quant = jax.lax.dot_general(
        onehot, embt_ref[...], (((1,), (0,)), ((), ())),
        preferred_element_type=jnp.float32,
        precision=jax.lax.Precision.HIGHEST)
    quant_ref[...] = quant

    diff = quant - x
    lb = jnp.sum(diff * diff).reshape(1, 1)

    @pl.when((p == 0) & (b == 0))
    def _():
        loss_ref[...] = jnp.zeros((1, 1), jnp.float32)

    loss_ref[...] += lb

    @pl.when((p == N_POS_BLOCKS - 1) & (b == N_BATCH - 1))
    def _():
        loss_ref[...] = loss_ref[...] * (1.0 / 1048576.0)


@functools.partial(jax.jit, static_argnames=("interpret",))
def _vq_call(xf, posf, emb, embt, interpret=False):
    n_rows = xf.shape[0]
    grid = (N_POS_BLOCKS, N_BATCH)
    return pl.pallas_call(
        _vq_body,
        grid=grid,
        in_specs=[
            pl.BlockSpec((BLOCK, DIM), lambda p, b: (b * N_POS_BLOCKS + p, 0)),
            pl.BlockSpec((BLOCK, DIM), lambda p, b: (p, 0)),
            pl.BlockSpec((DIM, N_EMBED), lambda p, b: (0, 0)),
            pl.BlockSpec((N_EMBED, DIM), lambda p, b: (0, 0)),
        ],
        out_specs=[
            pl.BlockSpec((BLOCK, 1), lambda p, b: (b * N_POS_BLOCKS + p, 0)),
            pl.BlockSpec((BLOCK, DIM), lambda p, b: (b * N_POS_BLOCKS + p, 0)),
            pl.BlockSpec((1, 1), lambda p, b: (0, 0)),
        ],
        out_shape=[
            jax.ShapeDtypeStruct((n_rows, 1), jnp.int32),
            jax.ShapeDtypeStruct((n_rows, DIM), jnp.float32),
            jax.ShapeDtypeStruct((1, 1), jnp.float32),
        ],
        interpret=interpret,
    )(xf, posf, emb, embt)


def kernel(input, embed, pos_weight, interpret=False):
    b, c, h, w = input.shape
    n_rows = b * c * h * w // DIM
    xf = input.reshape(n_rows, DIM)
    posf = pos_weight.reshape(c * h * w // DIM, DIM)
    embt = embed.T
    idx, quant, loss = _vq_call(xf, posf, embed, embt, interpret=interpret)
    return (quant.reshape(b, c, h, w),
            idx.reshape(b, c, h * w // DIM),
            loss[0, 0])


# speedup vs baseline: 1.7309x; 1.5896x over previous
import functools

import jax
import jax.numpy as jnp
from jax import lax
from jax.experimental import pallas as pl
from jax.experimental.pallas import tpu as pltpu
from jax.experimental.pallas import tpu_sc as plsc

DIM = 16
N_EMBED = 1024
BLOCK = 1024
N_POS_BLOCKS = 8
N_BATCH = 8

N_CORES = 2
N_SUBCORES = 16
N_WORKERS = N_CORES * N_SUBCORES
ROWS_PER_WORKER = 65536 // N_WORKERS
GATHER_CHUNK = 128
N_CHUNKS = ROWS_PER_WORKER // GATHER_CHUNK


def _vq_body(x_ref, pos_ref, emb_ref, idx_ref, loss_ref, pmm_ref):
    p = pl.program_id(0)
    b = pl.program_id(1)

    x = x_ref[...]
    pos = pos_ref[...]
    q = x + pos
    q = x + (q - x)

    emb = emb_ref[...]
    emb2 = emb + emb

    @pl.when(b == 0)
    def _():
        pmm_ref[...] = jax.lax.dot_general(
            pos, emb2, (((1,), (0,)), ((), ())),
            preferred_element_type=jnp.float32)

    mm2 = jax.lax.dot_general(
        q, emb2, (((1,), (0,)), ((), ())),
        preferred_element_type=jnp.float32)
    rowsum = jnp.sum(q * q, axis=1, keepdims=True)
    colsum = jnp.sum(emb * emb, axis=0, keepdims=True)
    dist = rowsum - mm2 + colsum

    m = jnp.min(dist, axis=1, keepdims=True)
    lanes = jax.lax.broadcasted_iota(jnp.int32, dist.shape, 1)
    idx = jnp.min(jnp.where(dist == m, lanes, jnp.int32(2**30)),
                  axis=1, keepdims=True)
    idx_ref[...] = idx

    selp2 = jnp.sum(jnp.where(lanes == idx, pmm_ref[...], 0.0),
                    axis=1, keepdims=True)
    rxs = jnp.sum(x * x, axis=1, keepdims=True)
    loss_rows = rxs + (m - rowsum) + selp2
    lb = jnp.sum(loss_rows).reshape(1, 1)

    @pl.when((p == 0) & (b == 0))
    def _():
        loss_ref[...] = jnp.zeros((1, 1), jnp.float32)

    loss_ref[...] += lb

    @pl.when((p == N_POS_BLOCKS - 1) & (b == N_BATCH - 1))
    def _():
        loss_ref[...] = loss_ref[...] * (1.0 / 1048576.0)


@jax.jit
def _vq_call(xf, posf, emb):
    n_rows = xf.shape[0]
    grid = (N_POS_BLOCKS, N_BATCH)
    return pl.pallas_call(
        _vq_body,
        grid=grid,
        in_specs=[
            pl.BlockSpec((BLOCK, DIM), lambda p, b: (b * N_POS_BLOCKS + p, 0)),
            pl.BlockSpec((BLOCK, DIM), lambda p, b: (p, 0)),
            pl.BlockSpec((DIM, N_EMBED), lambda p, b: (0, 0)),
        ],
        out_specs=[
            pl.BlockSpec((BLOCK, 1), lambda p, b: (b * N_POS_BLOCKS + p, 0)),
            pl.BlockSpec((1, 1), lambda p, b: (0, 0)),
        ],
        out_shape=[
            jax.ShapeDtypeStruct((n_rows, 1), jnp.int32),
            jax.ShapeDtypeStruct((1, 1), jnp.float32),
        ],
        scratch_shapes=[pltpu.VMEM((BLOCK, N_EMBED), jnp.float32)],
    )(xf, posf, emb)


def _gather_body(table_hbm, idx_hbm, out_hbm, idx_v, rows_v, sem):
    wid = lax.axis_index("s") * N_CORES + lax.axis_index("c")
    base = wid * ROWS_PER_WORKER
    pltpu.sync_copy(idx_hbm.at[wid], idx_v)
    for j in range(N_CHUNKS):
        pltpu.async_copy(table_hbm.at[idx_v.at[j]],
                         rows_v.at[pl.ds(j * GATHER_CHUNK, GATHER_CHUNK)],
                         sem)
    for _ in range(N_CHUNKS):
        pltpu.make_async_copy(
            table_hbm.at[idx_v.at[0]],
            rows_v.at[pl.ds(0, GATHER_CHUNK)], sem).wait()
    pltpu.sync_copy(rows_v, out_hbm.at[pl.ds(base, ROWS_PER_WORKER)])


@jax.jit
def _gather_call(table, idx):
    n_rows = idx.size
    idx3 = idx.reshape(N_WORKERS, N_CHUNKS, GATHER_CHUNK)
    return pl.kernel(
        _gather_body,
        out_type=jax.ShapeDtypeStruct((n_rows, DIM), jnp.float32),
        mesh=plsc.VectorSubcoreMesh(core_axis_name="c", subcore_axis_name="s"),
        scratch_types=[
            pltpu.VMEM((N_CHUNKS, GATHER_CHUNK), jnp.int32),
            pltpu.VMEM((ROWS_PER_WORKER, DIM), jnp.float32),
            pltpu.SemaphoreType.DMA,
        ],
        compiler_params=pltpu.CompilerParams(use_tc_tiling_on_sc=False),
    )(table, idx3)


def kernel(input, embed, pos_weight):
    b, c, h, w = input.shape
    n_rows = b * c * h * w // DIM
    xf = input.reshape(n_rows, DIM)
    posf = pos_weight.reshape(c * h * w // DIM, DIM)
    idx, loss = _vq_call(xf, posf, embed)
    quant = _gather_call(embed.T, idx.reshape(n_rows))
    return (quant.reshape(b, c, h, w),
            idx.reshape(b, c, h * w // DIM),
            loss[0, 0])


# speedup vs baseline: 1.7870x; 1.0324x over previous
import functools

import jax
import jax.numpy as jnp
from jax import lax
from jax.experimental import pallas as pl
from jax.experimental.pallas import tpu as pltpu
from jax.experimental.pallas import tpu_sc as plsc

DIM = 16
N_EMBED = 1024
BLOCK = 2048
N_POS_BLOCKS = 4
N_BATCH = 8

N_CORES = 2
N_SUBCORES = 16
N_WORKERS = N_CORES * N_SUBCORES
ROWS_PER_WORKER = 65536 // N_WORKERS
GATHER_CHUNK = 128
N_CHUNKS = ROWS_PER_WORKER // GATHER_CHUNK


def _vq_body(x_ref, pos_ref, emb_ref, idx_ref, loss_ref, pmm_ref):
    p = pl.program_id(0)
    b = pl.program_id(1)

    x = x_ref[...]
    pos = pos_ref[...]
    q = x + pos
    q = x + (q - x)

    emb = emb_ref[...]
    emb2 = emb + emb

    @pl.when(b == 0)
    def _():
        pmm_ref[...] = jax.lax.dot_general(
            pos, emb2, (((1,), (0,)), ((), ())),
            preferred_element_type=jnp.float32)

    mm2 = jax.lax.dot_general(
        q, emb2, (((1,), (0,)), ((), ())),
        preferred_element_type=jnp.float32)
    rowsum = jnp.sum(q * q, axis=1, keepdims=True)
    colsum = jnp.sum(emb * emb, axis=0, keepdims=True)
    dist = rowsum - mm2 + colsum

    m = jnp.min(dist, axis=1, keepdims=True)
    lanes = jax.lax.broadcasted_iota(jnp.int32, dist.shape, 1)
    idx = jnp.min(jnp.where(dist == m, lanes, jnp.int32(2**30)),
                  axis=1, keepdims=True)
    idx_ref[...] = idx

    selp2 = jnp.sum(jnp.where(lanes == idx, pmm_ref[...], 0.0),
                    axis=1, keepdims=True)
    rxs = jnp.sum(x * x, axis=1, keepdims=True)
    loss_rows = rxs + (m - rowsum) + selp2
    lb = jnp.sum(loss_rows).reshape(1, 1)

    @pl.when((p == 0) & (b == 0))
    def _():
        loss_ref[...] = jnp.zeros((1, 1), jnp.float32)

    loss_ref[...] += lb

    @pl.when((p == N_POS_BLOCKS - 1) & (b == N_BATCH - 1))
    def _():
        loss_ref[...] = loss_ref[...] * (1.0 / 1048576.0)


@jax.jit
def _vq_call(xf, posf, emb):
    n_rows = xf.shape[0]
    grid = (N_POS_BLOCKS, N_BATCH)
    return pl.pallas_call(
        _vq_body,
        grid=grid,
        in_specs=[
            pl.BlockSpec((BLOCK, DIM), lambda p, b: (b * N_POS_BLOCKS + p, 0)),
            pl.BlockSpec((BLOCK, DIM), lambda p, b: (p, 0)),
            pl.BlockSpec((DIM, N_EMBED), lambda p, b: (0, 0)),
        ],
        out_specs=[
            pl.BlockSpec((BLOCK, 1), lambda p, b: (b * N_POS_BLOCKS + p, 0)),
            pl.BlockSpec((1, 1), lambda p, b: (0, 0)),
        ],
        out_shape=[
            jax.ShapeDtypeStruct((n_rows, 1), jnp.int32),
            jax.ShapeDtypeStruct((1, 1), jnp.float32),
        ],
        scratch_shapes=[pltpu.VMEM((BLOCK, N_EMBED), jnp.float32)],
    )(xf, posf, emb)


def _gather_body(table_hbm, idx_hbm, out_hbm, idx_v, rows_v, sem):
    wid = lax.axis_index("s") * N_CORES + lax.axis_index("c")
    base = wid * ROWS_PER_WORKER
    pltpu.sync_copy(idx_hbm.at[wid], idx_v)
    for j in range(N_CHUNKS):
        pltpu.async_copy(table_hbm.at[idx_v.at[j]],
                         rows_v.at[pl.ds(j * GATHER_CHUNK, GATHER_CHUNK)],
                         sem)
    for _ in range(N_CHUNKS):
        pltpu.make_async_copy(
            table_hbm.at[idx_v.at[0]],
            rows_v.at[pl.ds(0, GATHER_CHUNK)], sem).wait()
    pltpu.sync_copy(rows_v, out_hbm.at[pl.ds(base, ROWS_PER_WORKER)])


@jax.jit
def _gather_call(table, idx):
    n_rows = idx.size
    idx3 = idx.reshape(N_WORKERS, N_CHUNKS, GATHER_CHUNK)
    return pl.kernel(
        _gather_body,
        out_type=jax.ShapeDtypeStruct((n_rows, DIM), jnp.float32),
        mesh=plsc.VectorSubcoreMesh(core_axis_name="c", subcore_axis_name="s"),
        scratch_types=[
            pltpu.VMEM((N_CHUNKS, GATHER_CHUNK), jnp.int32),
            pltpu.VMEM((ROWS_PER_WORKER, DIM), jnp.float32),
            pltpu.SemaphoreType.DMA,
        ],
        compiler_params=pltpu.CompilerParams(use_tc_tiling_on_sc=False),
    )(table, idx3)


def kernel(input, embed, pos_weight):
    b, c, h, w = input.shape
    n_rows = b * c * h * w // DIM
    xf = input.reshape(n_rows, DIM)
    posf = pos_weight.reshape(c * h * w // DIM, DIM)
    idx, loss = _vq_call(xf, posf, embed)
    quant = _gather_call(embed.T, idx.reshape(n_rows))
    return (quant.reshape(b, c, h, w),
            idx.reshape(b, c, h * w // DIM),
            loss[0, 0])
